# baseline (speedup 1.0000x reference)
import functools

import jax
import jax.numpy as jnp
from jax import lax
from jax.experimental import pallas as pl
from jax.experimental.pallas import tpu as pltpu
from jax.experimental.pallas import tpu_sc as plsc

EMBED_DIM = 128
BATCH = 16384

_NC = 2
_NS = 16
_NW = _NC * _NS
_B_PER_W = BATCH // _NW
_CHUNK = 32
_K = _B_PER_W // _CHUNK

_mesh = plsc.VectorSubcoreMesh(
    core_axis_name="c", subcore_axis_name="s",
    num_cores=_NC, num_subcores=_NS,
)


@functools.partial(
    pl.kernel,
    out_type=jax.ShapeDtypeStruct((_NW, _K, _CHUNK, EMBED_DIM), jnp.float32),
    mesh=_mesh,
    scratch_types=[
        pltpu.VMEM((_K, _CHUNK), jnp.int32),
        pltpu.VMEM((_K, _CHUNK, EMBED_DIM), jnp.float32),
        pltpu.VMEM_SHARED((1000, EMBED_DIM), jnp.float32),
        [pltpu.SemaphoreType.DMA] * _K,
        pltpu.SemaphoreType.DMA,
    ],
)
def _gather(t_hbm, table_hbm, out_hbm, idx_v, rows_v, tab_sh, gsems, ssem):
    cid = lax.axis_index("c")
    sid = lax.axis_index("s")
    wid = sid * _NC + cid
    pltpu.sync_copy(t_hbm.at[wid], idx_v)

    @pl.when(sid < 15)
    def _stage():
        pltpu.sync_copy(
            table_hbm.at[pl.ds(sid * 64, 64)],
            tab_sh.at[pl.ds(sid * 64, 64)],
        )

    @pl.when(sid == 15)
    def _stage_tail():
        pltpu.sync_copy(
            table_hbm.at[pl.ds(960, 40)],
            tab_sh.at[pl.ds(960, 40)],
        )

    plsc.subcore_barrier()
    gathers = [
        pltpu.async_copy(tab_sh.at[idx_v.at[j]], rows_v.at[j], gsems[j])
        for j in range(_K)
    ]
    stores = []
    for j in range(_K):
        gathers[j].wait()
        stores.append(pltpu.async_copy(rows_v.at[j], out_hbm.at[wid].at[j], ssem))
    for c in stores:
        c.wait()


def kernel(t, embeddings):
    t3 = t.reshape(_NW, _K, _CHUNK)
    out = _gather(t3, embeddings)
    return out.reshape(BATCH, EMBED_DIM)


# speedup vs baseline: 1.0068x; 1.0068x over previous
import functools

import jax
import jax.numpy as jnp
from jax import lax
from jax.experimental import pallas as pl
from jax.experimental.pallas import tpu as pltpu
from jax.experimental.pallas import tpu_sc as plsc

EMBED_DIM = 128
BATCH = 16384

_NC = 2
_NS = 16
_NW = _NC * _NS
_B_PER_W = BATCH // _NW
_CHUNK = 64
_K = _B_PER_W // _CHUNK

_mesh = plsc.VectorSubcoreMesh(
    core_axis_name="c", subcore_axis_name="s",
    num_cores=_NC, num_subcores=_NS,
)


@functools.partial(
    pl.kernel,
    out_type=jax.ShapeDtypeStruct((_NW, _K, _CHUNK, EMBED_DIM), jnp.float32),
    mesh=_mesh,
    scratch_types=[
        pltpu.VMEM((_K, _CHUNK), jnp.int32),
        pltpu.VMEM((_K, _CHUNK, EMBED_DIM), jnp.float32),
        pltpu.VMEM_SHARED((1000, EMBED_DIM), jnp.float32),
        [pltpu.SemaphoreType.DMA] * _K,
        pltpu.SemaphoreType.DMA,
    ],
)
def _gather(t_hbm, table_hbm, out_hbm, idx_v, rows_v, tab_sh, gsems, ssem):
    cid = lax.axis_index("c")
    sid = lax.axis_index("s")
    wid = sid * _NC + cid
    pltpu.sync_copy(t_hbm.at[wid], idx_v)

    @pl.when(sid < 15)
    def _stage():
        pltpu.sync_copy(
            table_hbm.at[pl.ds(sid * 64, 64)],
            tab_sh.at[pl.ds(sid * 64, 64)],
        )

    @pl.when(sid == 15)
    def _stage_tail():
        pltpu.sync_copy(
            table_hbm.at[pl.ds(960, 40)],
            tab_sh.at[pl.ds(960, 40)],
        )

    plsc.subcore_barrier()
    gathers = [
        pltpu.async_copy(tab_sh.at[idx_v.at[j]], rows_v.at[j], gsems[j])
        for j in range(_K)
    ]
    stores = []
    for j in range(_K):
        gathers[j].wait()
        stores.append(pltpu.async_copy(rows_v.at[j], out_hbm.at[wid].at[j], ssem))
    for c in stores:
        c.wait()


def kernel(t, embeddings):
    t3 = t.reshape(_NW, _K, _CHUNK)
    out = _gather(t3, embeddings)
    return out.reshape(BATCH, EMBED_DIM)


# speedup vs baseline: 1.0144x; 1.0076x over previous
import functools

import jax
import jax.numpy as jnp
from jax import lax
from jax.experimental import pallas as pl
from jax.experimental.pallas import tpu as pltpu
from jax.experimental.pallas import tpu_sc as plsc

EMBED_DIM = 128
BATCH = 16384

_NC = 2
_NS = 16
_NW = _NC * _NS
_B_PER_W = BATCH // _NW
_CHUNK = 64
_K = _B_PER_W // _CHUNK

_mesh = plsc.VectorSubcoreMesh(
    core_axis_name="c", subcore_axis_name="s",
    num_cores=_NC, num_subcores=_NS,
)


@functools.partial(
    pl.kernel,
    out_type=jax.ShapeDtypeStruct((_NW, _K, _CHUNK, EMBED_DIM), jnp.float32),
    mesh=_mesh,
    scratch_types=[
        pltpu.VMEM((_B_PER_W,), jnp.int32),
        pltpu.VMEM((_K, _CHUNK, EMBED_DIM), jnp.float32),
        pltpu.VMEM_SHARED((1000, EMBED_DIM), jnp.float32),
        [pltpu.SemaphoreType.DMA] * _K,
        pltpu.SemaphoreType.DMA,
    ],
)
def _gather(t_hbm, table_hbm, out_hbm, idx_v, rows_v, tab_sh, gsems, ssem):
    cid = lax.axis_index("c")
    sid = lax.axis_index("s")
    wid = sid * _NC + cid
    pltpu.sync_copy(t_hbm.at[pl.ds(wid * _B_PER_W, _B_PER_W)], idx_v)

    @pl.when(sid < 15)
    def _stage():
        pltpu.sync_copy(
            table_hbm.at[pl.ds(sid * 64, 64)],
            tab_sh.at[pl.ds(sid * 64, 64)],
        )

    @pl.when(sid == 15)
    def _stage_tail():
        pltpu.sync_copy(
            table_hbm.at[pl.ds(960, 40)],
            tab_sh.at[pl.ds(960, 40)],
        )

    plsc.subcore_barrier()
    gathers = [
        pltpu.async_copy(tab_sh.at[idx_v.at[pl.ds(j * _CHUNK, _CHUNK)]],
                         rows_v.at[j], gsems[j])
        for j in range(_K)
    ]
    stores = []
    for j in range(_K):
        gathers[j].wait()
        stores.append(pltpu.async_copy(rows_v.at[j], out_hbm.at[wid].at[j], ssem))
    for c in stores:
        c.wait()


def kernel(t, embeddings):
    out = _gather(t, embeddings)
    return out.reshape(BATCH, EMBED_DIM)
